# Initial kernel scaffold; baseline (speedup 1.0000x reference)
import jax
import jax.numpy as jnp
from jax.experimental import pallas as pl


def kernel(c):
    raise NotImplementedError("write your pallas kernel here")


# speedup vs baseline: 1.0843x; 1.0843x over previous
import functools

import jax
import jax.numpy as jnp
from jax import lax
from jax.experimental import pallas as pl

MIN_T = 2
MAX_T = 980
K = 1000
ROWS_PER_BLOCK = 256


def _threefry_bits(idx_u32):
    ks = (jnp.uint32(0), jnp.uint32(1), jnp.uint32(0x1BD11BDB))
    rotations = ((13, 15, 26, 6), (17, 29, 16, 24))
    x0 = jnp.zeros_like(idx_u32)
    x1 = idx_u32 + jnp.uint32(1)
    for i in range(5):
        for r in rotations[i % 2]:
            x0 = x0 + x1
            x1 = ((x1 << jnp.uint32(r)) | (x1 >> jnp.uint32(32 - r))) ^ x0
        x0 = x0 + ks[(i + 1) % 3]
        x1 = x1 + ks[(i + 2) % 3] + jnp.uint32(i + 1)
    return x0 ^ x1


def _sample_block(c_ref, o_ref, *, rows):
    x = c_ref[...]
    col = lax.broadcasted_iota(jnp.int32, (rows, K), 1)
    masked = (col < MIN_T) | (col >= MAX_T)
    logits = jnp.where(masked, x - 1e8, x)
    m = jnp.max(logits, axis=1, keepdims=True)
    e = jnp.exp(logits - m)
    s = jnp.sum(e, axis=1, keepdims=True)
    p = e / s + 1e-8
    lp = jnp.log(p)

    row = lax.broadcasted_iota(jnp.int32, (rows, K), 0)
    base = pl.program_id(0) * (rows * K)
    idx = (base + row * K + col).astype(jnp.uint32)
    bits = _threefry_bits(idx)

    fbits = (bits >> jnp.uint32(9)) | jnp.uint32(0x3F800000)
    f = lax.bitcast_convert_type(fbits, jnp.float32) - jnp.float32(1.0)
    tiny = jnp.float32(1.1754944e-38)
    u = f * (jnp.float32(1.0) - tiny) + tiny
    u = jnp.maximum(tiny, u)
    g = -jnp.log(-jnp.log(u))

    o_ref[...] = jnp.argmax(lp + g, axis=1).astype(jnp.int32)


@jax.jit
def kernel(c):
    n, l, k = c.shape
    rows = n * l
    x = c.reshape(rows, k)
    grid = rows // ROWS_PER_BLOCK
    out = pl.pallas_call(
        functools.partial(_sample_block, rows=ROWS_PER_BLOCK),
        grid=(grid,),
        in_specs=[pl.BlockSpec((ROWS_PER_BLOCK, k), lambda i: (i, 0))],
        out_specs=pl.BlockSpec((ROWS_PER_BLOCK,), lambda i: (i,)),
        out_shape=jax.ShapeDtypeStruct((rows,), jnp.int32),
    )(x)
    return out.reshape(n, l).astype(jnp.int64)


# speedup vs baseline: 1.1831x; 1.0911x over previous
import functools

import jax
import jax.numpy as jnp
from jax import lax
from jax.experimental import pallas as pl
from jax.experimental.pallas import tpu as pltpu
from jax.sharding import PartitionSpec as P

MIN_T = 2
MAX_T = 980
K = 1000
ROWS_PER_BLOCK = 256


def _threefry_bits(idx_u32):
    ks = (jnp.uint32(0), jnp.uint32(1), jnp.uint32(0x1BD11BDB))
    rotations = ((13, 15, 26, 6), (17, 29, 16, 24))
    x0 = jnp.zeros_like(idx_u32)
    x1 = idx_u32 + jnp.uint32(1)
    for i in range(5):
        for r in rotations[i % 2]:
            x0 = x0 + x1
            x1 = ((x1 << jnp.uint32(r)) | (x1 >> jnp.uint32(32 - r))) ^ x0
        x0 = x0 + ks[(i + 1) % 3]
        x1 = x1 + ks[(i + 2) % 3] + jnp.uint32(i + 1)
    return x0 ^ x1


def _sample_block(base_ref, c_ref, o_ref, *, rows):
    x = c_ref[...]
    col = lax.broadcasted_iota(jnp.int32, (rows, K), 1)
    masked = (col < MIN_T) | (col >= MAX_T)
    logits = jnp.where(masked, x - 1e8, x)
    m = jnp.max(logits, axis=1, keepdims=True)
    e = jnp.exp(logits - m)
    s = jnp.sum(e, axis=1, keepdims=True)
    p = e / s + 1e-8
    lp = jnp.log(p)

    row = lax.broadcasted_iota(jnp.int32, (rows, K), 0)
    base = (base_ref[0] + pl.program_id(0) * rows) * K
    idx = (base + row * K + col).astype(jnp.uint32)
    bits = _threefry_bits(idx)

    fbits = (bits >> jnp.uint32(9)) | jnp.uint32(0x3F800000)
    f = lax.bitcast_convert_type(fbits, jnp.float32) - jnp.float32(1.0)
    tiny = jnp.float32(1.1754944e-38)
    u = f * (jnp.float32(1.0) - tiny) + tiny
    u = jnp.maximum(tiny, u)
    g = -jnp.log(-jnp.log(u))

    o_ref[...] = jnp.argmax(lp + g, axis=1).astype(jnp.int32)


def _sample_rows(x, base_row):
    local_rows = x.shape[0]
    grid = local_rows // ROWS_PER_BLOCK
    return pl.pallas_call(
        functools.partial(_sample_block, rows=ROWS_PER_BLOCK),
        grid=(grid,),
        in_specs=[
            pl.BlockSpec(memory_space=pltpu.SMEM),
            pl.BlockSpec((ROWS_PER_BLOCK, K), lambda i: (i, 0)),
        ],
        out_specs=pl.BlockSpec((ROWS_PER_BLOCK,), lambda i: (i,)),
        out_shape=jax.ShapeDtypeStruct((local_rows,), jnp.int32),
    )(base_row.reshape(1).astype(jnp.int32), x)


@jax.jit
def kernel(c):
    n, l, k = c.shape
    rows = n * l
    x = c.reshape(rows, k)

    ndev = len(jax.devices())
    while ndev > 1 and rows % (ndev * ROWS_PER_BLOCK):
        ndev -= 1
    if ndev > 1:
        mesh = jax.make_mesh((ndev,), ("d",))
        local = rows // ndev
        x = jax.reshard(x, jax.sharding.NamedSharding(mesh, P("d", None)))

        def shard_fn(xs):
            base = lax.axis_index("d") * local
            return _sample_rows(xs, base)

        out = jax.shard_map(
            shard_fn,
            mesh=mesh,
            in_specs=P("d", None),
            out_specs=P("d"),
            check_vma=False,
        )(x)
    else:
        out = _sample_rows(x, jnp.int32(0))
    return out.reshape(n, l).astype(jnp.int64)
